# baseline (speedup 1.0000x reference)
import functools

import jax
import jax.numpy as jnp
from jax import lax
from jax.experimental import pallas as pl
from jax.experimental.pallas import tpu as pltpu
from jax.experimental.pallas import tpu_sc as plsc

NC = 2
NS = 16
NW = NC * NS
G = 128
K = 5


@functools.lru_cache(maxsize=None)
def _make_gather(V, D, B):
    assert B % (NW * G) == 0
    ng = B // (NW * G)
    assert ng % K == 0 and ng // K >= 4 and (ng // K) % 2 == 0
    nsuper = ng // K
    rows_per_w = ng * G

    mesh = plsc.VectorSubcoreMesh(core_axis_name="c", subcore_axis_name="s")

    @functools.partial(
        pl.kernel,
        out_type=jax.ShapeDtypeStruct((B, D), jnp.float32),
        mesh=mesh,
        compiler_params=pltpu.CompilerParams(use_tc_tiling_on_sc=False),
        scratch_types=[
            pltpu.VMEM((ng, G), jnp.int32),
            pltpu.VMEM((2 * K, G, D), jnp.float32),
            pltpu.SemaphoreType.DMA,
            pltpu.SemaphoreType.DMA,
            pltpu.SemaphoreType.DMA,
        ],
    )
    def gather_kernel(table_hbm, idx_hbm, out_hbm, idx_v, rows_v, g0sem, g1sem, ssem):
        wid = lax.axis_index("s") * NC + lax.axis_index("c")
        base = wid * rows_per_w
        pltpu.sync_copy(idx_hbm.at[wid], idx_v)
        gsems = (g0sem, g1sem)

        def fire_gathers(g0, parity):
            for b in range(K):
                pltpu.async_copy(
                    table_hbm.at[idx_v.at[g0 + b]],
                    rows_v.at[parity * K + b],
                    gsems[parity],
                )

        def drain_gathers(g0, parity):
            for b in range(K):
                pltpu.make_async_copy(
                    table_hbm.at[idx_v.at[g0 + b]],
                    rows_v.at[parity * K + b],
                    gsems[parity],
                ).wait()

        def fire_stores(g0, parity):
            for b in range(K):
                pltpu.async_copy(
                    rows_v.at[parity * K + b],
                    out_hbm.at[pl.ds(base + (g0 + b) * G, G)],
                    ssem,
                )

        def wait_stores(g0, parity):
            for b in range(K):
                pltpu.make_async_copy(
                    rows_v.at[parity * K + b],
                    out_hbm.at[pl.ds(base + (g0 + b) * G, G)],
                    ssem,
                ).wait()

        def steady_step(g0, parity):
            wait_stores(g0 - K, 1 - parity)
            fire_gathers(g0 + K, 1 - parity)
            drain_gathers(g0, parity)
            fire_stores(g0, parity)

        fire_gathers(0, 0)
        fire_gathers(K, 1)
        drain_gathers(0, 0)
        fire_stores(0, 0)

        @pl.loop(0, (nsuper - 2) // 2)
        def _(p):
            g0 = (2 * p + 1) * K
            steady_step(g0, 1)
            steady_step(g0 + K, 0)

        last = nsuper - 1
        wait_stores((last - 1) * K, (last - 1) % 2)
        drain_gathers(last * K, last % 2)
        fire_stores(last * K, last % 2)
        wait_stores(last * K, last % 2)

    return gather_kernel


def kernel(sequence, src_word_table):
    batch, seq_len = sequence.shape
    vocab, emsize = src_word_table.shape
    total = batch * seq_len
    idx = sequence.reshape(NW, total // (NW * G), G)
    out = _make_gather(vocab, emsize, total)(src_word_table, idx)
    return out.reshape(batch, seq_len, emsize)


# speedup vs baseline: 1.0013x; 1.0013x over previous
import functools

import jax
import jax.numpy as jnp
from jax import lax
from jax.experimental import pallas as pl
from jax.experimental.pallas import tpu as pltpu
from jax.experimental.pallas import tpu_sc as plsc

NC = 2
NS = 16
NW = NC * NS
G = 512


@functools.lru_cache(maxsize=None)
def _make_gather(V, D, B):
    assert B % (NW * G) == 0
    ng = B // (NW * G)
    assert ng >= 4 and ng % 2 == 0
    mesh = plsc.VectorSubcoreMesh(core_axis_name="c", subcore_axis_name="s")

    @functools.partial(
        pl.kernel,
        out_type=jax.ShapeDtypeStruct((B // G, G, D), jnp.float32),
        mesh=mesh,
        compiler_params=pltpu.CompilerParams(use_tc_tiling_on_sc=False),
        scratch_types=[
            pltpu.VMEM((ng, G), jnp.int32),
            pltpu.VMEM((2, G, D), jnp.float32),
            pltpu.SemaphoreType.DMA,
            pltpu.SemaphoreType.DMA,
            pltpu.SemaphoreType.DMA,
        ],
    )
    def gather_kernel(table_hbm, idx_hbm, out_hbm, idx_v, rows_v, g0sem, g1sem, ssem):
        wid = lax.axis_index("s") * NC + lax.axis_index("c")
        pltpu.sync_copy(idx_hbm.at[wid], idx_v)
        gsems = (g0sem, g1sem)

        def fire_gather(g, parity):
            pltpu.async_copy(
                table_hbm.at[idx_v.at[g]], rows_v.at[parity], gsems[parity])

        def drain_gather(g, parity):
            pltpu.make_async_copy(
                table_hbm.at[idx_v.at[g]], rows_v.at[parity], gsems[parity]
            ).wait()

        def fire_store(g, parity):
            pltpu.async_copy(rows_v.at[parity], out_hbm.at[wid * ng + g], ssem)

        def wait_store(g, parity):
            pltpu.make_async_copy(
                rows_v.at[parity], out_hbm.at[wid * ng + g], ssem
            ).wait()

        def steady_step(g, parity):
            wait_store(g - 1, 1 - parity)
            fire_gather(g + 1, 1 - parity)
            drain_gather(g, parity)
            fire_store(g, parity)

        fire_gather(0, 0)
        fire_gather(1, 1)
        drain_gather(0, 0)
        fire_store(0, 0)

        @pl.loop(0, (ng - 2) // 2)
        def _(p):
            g = 2 * p + 1
            steady_step(g, 1)
            steady_step(g + 1, 0)

        last = ng - 1
        wait_store(last - 1, (last - 1) % 2)
        drain_gather(last, last % 2)
        fire_store(last, last % 2)
        wait_store(last, last % 2)

    return gather_kernel


def kernel(sequence, src_word_table):
    batch, seq_len = sequence.shape
    vocab, emsize = src_word_table.shape
    total = batch * seq_len
    idx = sequence.reshape(NW, total // (NW * G), G)
    out = _make_gather(vocab, emsize, total)(src_word_table, idx)
    return out.reshape(batch, seq_len, emsize)
